# baseline (speedup 1.0000x reference)
import functools

import jax
import jax.numpy as jnp
from jax import lax
from jax.experimental import pallas as pl
from jax.experimental.pallas import tpu as pltpu
from jax.experimental.pallas import tpu_sc as plsc

N = 10000
E = 320000
D = 128
OUT = 40

NC = 2
NS = 16
NW = NC * NS
CH = 80
NCHUNK = 128
EPW = NCHUNK * CH
E_PAD = NW * EPW
N_PAD = 10240
N_TRASH = 240
RPT = N_PAD // NS
DEG_W = 16

_mesh = plsc.VectorSubcoreMesh(core_axis_name="c", subcore_axis_name="s")


@functools.partial(
    pl.kernel,
    out_type=jax.ShapeDtypeStruct((NC, N_PAD, DEG_W), jnp.float32),
    mesh=_mesh,
    scratch_types=[
        pltpu.VMEM((NCHUNK, CH), jnp.int32),
        pltpu.VMEM((CH, DEG_W), jnp.float32),
        pltpu.VMEM_SHARED((N_PAD, DEG_W), jnp.float32),
        pltpu.SemaphoreType.DMA,
    ],
)
def _deg_kernel(dst_hbm, out_hbm, dst_v, ones_v, acc, sem):
    cid = lax.axis_index("c")
    sid = lax.axis_index("s")
    w = cid * NS + sid

    zeros16 = jnp.zeros((16,), jnp.float32)
    ones16 = jnp.ones((16,), jnp.float32)

    def zrow(i, _):
        ones_v[i, :] = zeros16
        return 0

    lax.fori_loop(0, CH, zrow, 0)
    pltpu.sync_copy(dst_hbm.at[w], dst_v)
    for k in range(RPT // CH):
        pltpu.sync_copy(ones_v, acc.at[pl.ds(sid * RPT + k * CH, CH)])

    def fill(i, _):
        ones_v[i, :] = ones16
        return 0

    lax.fori_loop(0, CH, fill, 0)
    plsc.subcore_barrier()

    def body(c, _):
        pltpu.sync_copy(ones_v, acc.at[dst_v.at[c]], add=True)
        return 0

    lax.fori_loop(0, NCHUNK, body, 0)
    plsc.subcore_barrier()
    pltpu.sync_copy(acc.at[pl.ds(sid * RPT, RPT)],
                    out_hbm.at[cid, pl.ds(sid * RPT, RPT)])


@functools.partial(
    pl.kernel,
    out_type=jax.ShapeDtypeStruct((NC, N_PAD, D), jnp.float32),
    mesh=_mesh,
    scratch_types=[
        pltpu.VMEM((NCHUNK, CH), jnp.int32),
        pltpu.VMEM((CH, D), jnp.float32),
        pltpu.VMEM((CH, D), jnp.float32),
        [pltpu.VMEM((CH,), jnp.int32) for _ in range(4)],
        pltpu.VMEM_SHARED((N_PAD, D), jnp.float32),
        pltpu.SemaphoreType.DMA,
        pltpu.SemaphoreType.DMA,
        [pltpu.SemaphoreType.DMA for _ in range(4)],
    ],
)
def _spmm_kernel(ms_hbm, src_hbm, dst_hbm, out_hbm, src_v, rows0, rows1,
                 dring, acc, semr0, semr1, semi):
    cid = lax.axis_index("c")
    sid = lax.axis_index("s")
    w = cid * NS + sid
    wbase = w * NCHUNK

    zeros16 = jnp.zeros((16,), jnp.float32)

    def zrow(i, _):
        for j in range(D // 16):
            rows0[i, pl.ds(j * 16, 16)] = zeros16
        return 0

    lax.fori_loop(0, CH, zrow, 0)
    pltpu.sync_copy(src_hbm.at[w], src_v)
    for k in range(RPT // CH):
        pltpu.sync_copy(rows0, acc.at[pl.ds(sid * RPT + k * CH, CH)])
    plsc.subcore_barrier()

    for j in range(4):
        pltpu.async_copy(dst_hbm.at[wbase + j], dring[j], semi[j])
    pltpu.async_copy(ms_hbm.at[src_v.at[0]], rows0, semr0)

    rbufs = (rows0, rows1)
    rsems = (semr0, semr1)

    def q_body(q, _):
        c = 4 * q
        for j in range(4):
            cj = c + j
            rows, semr = rbufs[j % 2], rsems[j % 2]
            orows, osemr = rbufs[(j + 1) % 2], rsems[(j + 1) % 2]
            pltpu.make_async_copy(ms_hbm.at[src_v.at[cj]], rows, semr).wait()

            @pl.when(cj + 1 < NCHUNK)
            def _():
                pltpu.async_copy(ms_hbm.at[src_v.at[cj + 1]], orows, osemr)

            pltpu.make_async_copy(dst_hbm.at[wbase + cj], dring[j],
                                  semi[j]).wait()
            pltpu.sync_copy(rows, acc.at[dring[j]], add=True)

            @pl.when(cj + 4 < NCHUNK)
            def _():
                pltpu.async_copy(dst_hbm.at[wbase + cj + 4], dring[j],
                                 semi[j])

        return 0

    lax.fori_loop(0, NCHUNK // 4, q_body, 0)
    plsc.subcore_barrier()
    pltpu.sync_copy(acc.at[pl.ds(sid * RPT, RPT)],
                    out_hbm.at[cid, pl.ds(sid * RPT, RPT)])


_RB = 1000


def _dis_from_parts(deg_ref):
    deg = deg_ref[0, :, 0] + deg_ref[1, :, 0] + 1.0
    return lax.rsqrt(deg)


def _tc1_body(x_ref, w1_ref, deg_ref, ms_ref):
    dis = _dis_from_parts(deg_ref)
    h = jnp.dot(x_ref[...], w1_ref[...], preferred_element_type=jnp.float32)
    ms_ref[...] = h * dis[:, None]


def _tc2_body(p_ref, ms1_ref, deg_ref, w2_ref, b1_ref, ms2_ref):
    dis = _dis_from_parts(deg_ref)
    s = p_ref[0] + p_ref[1] + ms1_ref[...]
    h1 = jnp.maximum(s * dis[:, None] + b1_ref[...], 0.0)
    h2 = jnp.dot(h1, w2_ref[...], preferred_element_type=jnp.float32)
    ms2_ref[...] = h2 * dis[:, None]


def _tc3_body(p_ref, ms2_ref, deg_ref, b2_ref, wl_ref, bl_ref, out_ref,
              emb_ref):
    dis = _dis_from_parts(deg_ref)
    s = p_ref[0] + p_ref[1] + ms2_ref[...]
    emb = s * dis[:, None] + b2_ref[...]
    emb_ref[...] = emb
    logits = jnp.dot(emb, wl_ref[...], preferred_element_type=jnp.float32)
    logits = logits + bl_ref[...]
    m = jnp.max(logits, axis=1, keepdims=True)
    z = logits - m
    lse = jnp.log(jnp.sum(jnp.exp(z), axis=1, keepdims=True))
    out_ref[...] = z - lse


def kernel(x, edge_index, W1, b1, W2, b2, Wl, bl):
    npad = E_PAD - E
    srcp = jnp.concatenate(
        [edge_index[0], jnp.zeros((npad,), edge_index.dtype)])
    trash = N + jnp.arange(npad, dtype=edge_index.dtype) % N_TRASH
    dstp = jnp.concatenate([edge_index[1], trash])
    src = srcp.reshape(NW, NCHUNK, CH)
    dst = dstp.reshape(NW * NCHUNK, CH)
    dst3 = dstp.reshape(NW, NCHUNK, CH)

    deg_parts = _deg_kernel(dst3)

    grid = (N // _RB,)
    full = lambda i: (0, 0)
    rowb = lambda i: (i, 0)
    degb = lambda i: (0, i, 0)

    deg_spec = pl.BlockSpec((NC, _RB, DEG_W), degb)
    part_spec = pl.BlockSpec((NC, _RB, D), degb)
    feat_spec = pl.BlockSpec((_RB, D), rowb)

    ms1 = pl.pallas_call(
        _tc1_body,
        grid=grid,
        in_specs=[feat_spec, pl.BlockSpec((D, D), full), deg_spec],
        out_specs=feat_spec,
        out_shape=jax.ShapeDtypeStruct((N, D), jnp.float32),
    )(x, W1, deg_parts[:, :N, :])

    p1 = _spmm_kernel(ms1, src, dst)

    ms2 = pl.pallas_call(
        _tc2_body,
        grid=grid,
        in_specs=[part_spec, feat_spec, deg_spec,
                  pl.BlockSpec((D, D), full), pl.BlockSpec((1, D), full)],
        out_specs=feat_spec,
        out_shape=jax.ShapeDtypeStruct((N, D), jnp.float32),
    )(p1[:, :N, :], ms1, deg_parts[:, :N, :], W2, b1.reshape(1, D))

    p2 = _spmm_kernel(ms2, src, dst)

    out, emb = pl.pallas_call(
        _tc3_body,
        grid=grid,
        in_specs=[part_spec, feat_spec, deg_spec,
                  pl.BlockSpec((1, D), full), pl.BlockSpec((D, OUT), full),
                  pl.BlockSpec((1, OUT), full)],
        out_specs=[pl.BlockSpec((_RB, OUT), rowb), feat_spec],
        out_shape=[jax.ShapeDtypeStruct((N, OUT), jnp.float32),
                   jax.ShapeDtypeStruct((N, D), jnp.float32)],
    )(p2[:, :N, :], ms2, deg_parts[:, :N, :], b2.reshape(1, D), Wl,
      bl.reshape(1, OUT))

    return (out, emb)


# speedup vs baseline: 1.0261x; 1.0261x over previous
import functools

import jax
import jax.numpy as jnp
from jax import lax
from jax.experimental import pallas as pl
from jax.experimental.pallas import tpu as pltpu
from jax.experimental.pallas import tpu_sc as plsc

N = 10000
E = 320000
D = 128
OUT = 40

NC = 2
NS = 16
NW = NC * NS
CH = 80
NCHUNK = 128
EPW = NCHUNK * CH
E_PAD = NW * EPW
N_PAD = 10240
N_TRASH = 240
RPT = N_PAD // NS
DEG_W = 16

_mesh = plsc.VectorSubcoreMesh(core_axis_name="c", subcore_axis_name="s")


@functools.partial(
    pl.kernel,
    out_type=jax.ShapeDtypeStruct((NC, N_PAD, DEG_W), jnp.float32),
    mesh=_mesh,
    scratch_types=[
        pltpu.VMEM((NCHUNK, CH), jnp.int32),
        pltpu.VMEM((CH, DEG_W), jnp.float32),
        pltpu.VMEM_SHARED((N_PAD, DEG_W), jnp.float32),
        pltpu.SemaphoreType.DMA,
    ],
)
def _deg_kernel(dst_hbm, out_hbm, dst_v, ones_v, acc, sem):
    cid = lax.axis_index("c")
    sid = lax.axis_index("s")
    w = cid * NS + sid

    zeros16 = jnp.zeros((16,), jnp.float32)
    ones16 = jnp.ones((16,), jnp.float32)

    def zrow(i, _):
        ones_v[i, :] = zeros16
        return 0

    lax.fori_loop(0, CH, zrow, 0)
    pltpu.sync_copy(dst_hbm.at[w], dst_v)
    for k in range(RPT // CH):
        pltpu.sync_copy(ones_v, acc.at[pl.ds(sid * RPT + k * CH, CH)])

    def fill(i, _):
        ones_v[i, :] = ones16
        return 0

    lax.fori_loop(0, CH, fill, 0)
    plsc.subcore_barrier()

    def body(c, _):
        pltpu.sync_copy(ones_v, acc.at[dst_v.at[c]], add=True)
        return 0

    lax.fori_loop(0, NCHUNK, body, 0)
    plsc.subcore_barrier()
    pltpu.sync_copy(acc.at[pl.ds(sid * RPT, RPT)],
                    out_hbm.at[cid, pl.ds(sid * RPT, RPT)])


@functools.partial(
    pl.kernel,
    out_type=jax.ShapeDtypeStruct((NC, N_PAD, D), jnp.float32),
    mesh=_mesh,
    scratch_types=[
        pltpu.VMEM((NCHUNK, CH), jnp.int32),
        pltpu.VMEM((CH, D), jnp.float32),
        pltpu.VMEM((CH, D), jnp.float32),
        [pltpu.VMEM((CH,), jnp.int32) for _ in range(4)],
        pltpu.VMEM_SHARED((N_PAD, D), jnp.float32),
        pltpu.SemaphoreType.DMA,
        pltpu.SemaphoreType.DMA,
        [pltpu.SemaphoreType.DMA for _ in range(4)],
    ],
)
def _spmm_kernel(ms_hbm, src_hbm, dst_hbm, out_hbm, src_v, rows0, rows1,
                 dring, acc, semr0, semr1, semi):
    cid = lax.axis_index("c")
    sid = lax.axis_index("s")
    w = cid * NS + sid
    wbase = w * NCHUNK

    zeros16 = jnp.zeros((16,), jnp.float32)

    def zrow(i, _):
        for j in range(D // 16):
            rows0[i, pl.ds(j * 16, 16)] = zeros16
        return 0

    lax.fori_loop(0, CH, zrow, 0)
    pltpu.sync_copy(src_hbm.at[w], src_v)
    for k in range(RPT // CH):
        pltpu.sync_copy(rows0, acc.at[pl.ds(sid * RPT + k * CH, CH)])
    plsc.subcore_barrier()

    for j in range(4):
        pltpu.async_copy(dst_hbm.at[wbase + j], dring[j], semi[j])
    pltpu.async_copy(ms_hbm.at[src_v.at[0]], rows0, semr0)

    rbufs = (rows0, rows1)
    rsems = (semr0, semr1)

    def q_body(q, _):
        c = 4 * q
        for j in range(4):
            cj = c + j
            rows, semr = rbufs[j % 2], rsems[j % 2]
            orows, osemr = rbufs[(j + 1) % 2], rsems[(j + 1) % 2]
            pltpu.make_async_copy(ms_hbm.at[src_v.at[cj]], rows, semr).wait()
            pltpu.async_copy(ms_hbm.at[src_v.at[cj + 1]], orows, osemr)
            pltpu.make_async_copy(dst_hbm.at[wbase + cj], dring[j],
                                  semi[j]).wait()
            pltpu.sync_copy(rows, acc.at[dring[j]], add=True)
            pltpu.async_copy(dst_hbm.at[wbase + cj + 4], dring[j], semi[j])

        return 0

    lax.fori_loop(0, (NCHUNK - 4) // 4, q_body, 0)
    for cc in range(NCHUNK - 4, NCHUNK):
        rows, semr = rbufs[cc % 2], rsems[cc % 2]
        pltpu.make_async_copy(ms_hbm.at[src_v.at[cc]], rows, semr).wait()
        if cc + 1 < NCHUNK:
            pltpu.async_copy(ms_hbm.at[src_v.at[cc + 1]],
                             rbufs[(cc + 1) % 2], rsems[(cc + 1) % 2])
        pltpu.make_async_copy(dst_hbm.at[wbase + cc], dring[cc % 4],
                              semi[cc % 4]).wait()
        pltpu.sync_copy(rows, acc.at[dring[cc % 4]], add=True)
    plsc.subcore_barrier()
    pltpu.sync_copy(acc.at[pl.ds(sid * RPT, RPT)],
                    out_hbm.at[cid, pl.ds(sid * RPT, RPT)])


_RB = 1000


def _dis_from_parts(deg_ref):
    deg = deg_ref[0, :, 0] + deg_ref[1, :, 0] + 1.0
    return lax.rsqrt(deg)


def _tc1_body(x_ref, w1_ref, deg_ref, ms_ref):
    dis = _dis_from_parts(deg_ref)
    h = jnp.dot(x_ref[...], w1_ref[...], preferred_element_type=jnp.float32)
    ms_ref[...] = h * dis[:, None]


def _tc2_body(p_ref, ms1_ref, deg_ref, w2_ref, b1_ref, ms2_ref):
    dis = _dis_from_parts(deg_ref)
    s = p_ref[0] + p_ref[1] + ms1_ref[...]
    h1 = jnp.maximum(s * dis[:, None] + b1_ref[...], 0.0)
    h2 = jnp.dot(h1, w2_ref[...], preferred_element_type=jnp.float32)
    ms2_ref[...] = h2 * dis[:, None]


def _tc3_body(p_ref, ms2_ref, deg_ref, b2_ref, wl_ref, bl_ref, out_ref,
              emb_ref):
    dis = _dis_from_parts(deg_ref)
    s = p_ref[0] + p_ref[1] + ms2_ref[...]
    emb = s * dis[:, None] + b2_ref[...]
    emb_ref[...] = emb
    logits = jnp.dot(emb, wl_ref[...], preferred_element_type=jnp.float32)
    logits = logits + bl_ref[...]
    m = jnp.max(logits, axis=1, keepdims=True)
    z = logits - m
    lse = jnp.log(jnp.sum(jnp.exp(z), axis=1, keepdims=True))
    out_ref[...] = z - lse


def kernel(x, edge_index, W1, b1, W2, b2, Wl, bl):
    npad = E_PAD - E
    srcp = jnp.concatenate(
        [edge_index[0], jnp.zeros((npad,), edge_index.dtype)])
    trash = N + jnp.arange(npad, dtype=edge_index.dtype) % N_TRASH
    dstp = jnp.concatenate([edge_index[1], trash])
    src = srcp.reshape(NW, NCHUNK, CH)
    dst = dstp.reshape(NW * NCHUNK, CH)
    dst3 = dstp.reshape(NW, NCHUNK, CH)

    deg_parts = _deg_kernel(dst3)

    grid = (N // _RB,)
    full = lambda i: (0, 0)
    rowb = lambda i: (i, 0)
    degb = lambda i: (0, i, 0)

    deg_spec = pl.BlockSpec((NC, _RB, DEG_W), degb)
    part_spec = pl.BlockSpec((NC, _RB, D), degb)
    feat_spec = pl.BlockSpec((_RB, D), rowb)

    ms1 = pl.pallas_call(
        _tc1_body,
        grid=grid,
        in_specs=[feat_spec, pl.BlockSpec((D, D), full), deg_spec],
        out_specs=feat_spec,
        out_shape=jax.ShapeDtypeStruct((N, D), jnp.float32),
    )(x, W1, deg_parts[:, :N, :])

    p1 = _spmm_kernel(ms1, src, dst)

    ms2 = pl.pallas_call(
        _tc2_body,
        grid=grid,
        in_specs=[part_spec, feat_spec, deg_spec,
                  pl.BlockSpec((D, D), full), pl.BlockSpec((1, D), full)],
        out_specs=feat_spec,
        out_shape=jax.ShapeDtypeStruct((N, D), jnp.float32),
    )(p1[:, :N, :], ms1, deg_parts[:, :N, :], W2, b1.reshape(1, D))

    p2 = _spmm_kernel(ms2, src, dst)

    out, emb = pl.pallas_call(
        _tc3_body,
        grid=grid,
        in_specs=[part_spec, feat_spec, deg_spec,
                  pl.BlockSpec((1, D), full), pl.BlockSpec((D, OUT), full),
                  pl.BlockSpec((1, OUT), full)],
        out_specs=[pl.BlockSpec((_RB, OUT), rowb), feat_spec],
        out_shape=[jax.ShapeDtypeStruct((N, OUT), jnp.float32),
                   jax.ShapeDtypeStruct((N, D), jnp.float32)],
    )(p2[:, :N, :], ms2, deg_parts[:, :N, :], b2.reshape(1, D), Wl,
      bl.reshape(1, OUT))

    return (out, emb)


# speedup vs baseline: 2.8067x; 2.7352x over previous
import functools

import jax
import jax.numpy as jnp
from jax import lax
from jax.experimental import pallas as pl
from jax.experimental.pallas import tpu as pltpu
from jax.experimental.pallas import tpu_sc as plsc

N = 10000
E = 320000
D = 128
OUT = 40

NC = 2
NS = 16
NW = NC * NS
CH = 80
NCHUNK = 128
EPW = NCHUNK * CH
E_PAD = NW * EPW
N_PAD = 10240
N_TRASH = 240
RPT = N_PAD // NS
DEG_W = 16

_mesh = plsc.VectorSubcoreMesh(core_axis_name="c", subcore_axis_name="s")


@functools.partial(
    pl.kernel,
    out_type=jax.ShapeDtypeStruct((NC, N_PAD, DEG_W), jnp.float32),
    mesh=_mesh,
    scratch_types=[
        pltpu.VMEM((NCHUNK, CH), jnp.int32),
        pltpu.VMEM((CH, DEG_W), jnp.float32),
        pltpu.VMEM_SHARED((N_PAD, DEG_W), jnp.float32),
        pltpu.SemaphoreType.DMA,
    ],
)
def _deg_kernel(dst_hbm, out_hbm, dst_v, ones_v, acc, sem):
    cid = lax.axis_index("c")
    sid = lax.axis_index("s")
    w = cid * NS + sid

    zeros16 = jnp.zeros((16,), jnp.float32)
    ones16 = jnp.ones((16,), jnp.float32)

    def zrow(i, _):
        ones_v[i, :] = zeros16
        return 0

    lax.fori_loop(0, CH, zrow, 0)
    pltpu.sync_copy(dst_hbm.at[w], dst_v)
    for k in range(RPT // CH):
        pltpu.sync_copy(ones_v, acc.at[pl.ds(sid * RPT + k * CH, CH)])

    def fill(i, _):
        ones_v[i, :] = ones16
        return 0

    lax.fori_loop(0, CH, fill, 0)
    plsc.subcore_barrier()

    def body(c, _):
        pltpu.sync_copy(ones_v, acc.at[dst_v.at[c]], add=True)
        return 0

    lax.fori_loop(0, NCHUNK, body, 0)
    plsc.subcore_barrier()
    pltpu.sync_copy(acc.at[pl.ds(sid * RPT, RPT)],
                    out_hbm.at[cid, pl.ds(sid * RPT, RPT)])


@functools.partial(
    pl.kernel,
    out_type=jax.ShapeDtypeStruct((NC, N_PAD, D), jnp.float32),
    mesh=_mesh,
    scratch_types=[
        pltpu.VMEM((NCHUNK, CH), jnp.int32),
        pltpu.VMEM((CH, D), jnp.float32),
        pltpu.VMEM((CH, D), jnp.float32),
        [pltpu.VMEM((CH,), jnp.int32) for _ in range(4)],
        pltpu.VMEM_SHARED((N_PAD, D), jnp.float32),
        pltpu.SemaphoreType.DMA,
        pltpu.SemaphoreType.DMA,
        [pltpu.SemaphoreType.DMA for _ in range(4)],
    ],
)
def _spmm_kernel(ms_hbm, src_hbm, dst_hbm, out_hbm, src_v, rows0, rows1,
                 dring, acc, semr0, semr1, semi):
    cid = lax.axis_index("c")
    sid = lax.axis_index("s")
    w = cid * NS + sid
    wbase = w * NCHUNK

    zeros16 = jnp.zeros((16,), jnp.float32)

    def zrow(i, _):
        for j in range(D // 16):
            rows0[i, pl.ds(j * 16, 16)] = zeros16
        return 0

    lax.fori_loop(0, CH, zrow, 0)
    pltpu.sync_copy(src_hbm.at[w], src_v)
    for k in range(RPT // CH):
        pltpu.sync_copy(rows0, acc.at[pl.ds(sid * RPT + k * CH, CH)])
    plsc.subcore_barrier()

    for j in range(4):
        pltpu.async_copy(dst_hbm.at[wbase + j], dring[j], semi[j])
    pltpu.async_copy(ms_hbm.at[src_v.at[0]], rows0, semr0)

    rbufs = (rows0, rows1)
    rsems = (semr0, semr1)

    def q_body(q, _):
        c = 4 * q
        for j in range(4):
            cj = c + j
            rows, semr = rbufs[j % 2], rsems[j % 2]
            orows, osemr = rbufs[(j + 1) % 2], rsems[(j + 1) % 2]
            pltpu.make_async_copy(ms_hbm.at[src_v.at[cj]], rows, semr).wait()
            pltpu.async_copy(ms_hbm.at[src_v.at[cj + 1]], orows, osemr)
            pltpu.make_async_copy(dst_hbm.at[wbase + cj], dring[j],
                                  semi[j]).wait()
            pltpu.sync_copy(rows, acc.at[dring[j]], add=True)
            pltpu.async_copy(dst_hbm.at[wbase + cj + 4], dring[j], semi[j])

        return 0

    lax.fori_loop(0, (NCHUNK - 4) // 4, q_body, 0)
    for cc in range(NCHUNK - 4, NCHUNK):
        rows, semr = rbufs[cc % 2], rsems[cc % 2]
        pltpu.make_async_copy(ms_hbm.at[src_v.at[cc]], rows, semr).wait()
        if cc + 1 < NCHUNK:
            pltpu.async_copy(ms_hbm.at[src_v.at[cc + 1]],
                             rbufs[(cc + 1) % 2], rsems[(cc + 1) % 2])
        pltpu.make_async_copy(dst_hbm.at[wbase + cc], dring[cc % 4],
                              semi[cc % 4]).wait()
        pltpu.sync_copy(rows, acc.at[dring[cc % 4]], add=True)
    plsc.subcore_barrier()
    pltpu.sync_copy(acc.at[pl.ds(sid * RPT, RPT)],
                    out_hbm.at[cid, pl.ds(sid * RPT, RPT)])


_RB = 1000


def _dis_from_parts(deg_ref):
    deg = deg_ref[0, :, 0] + deg_ref[1, :, 0] + 1.0
    return lax.rsqrt(deg)


def _tc1_body(x_ref, w1_ref, deg_ref, ms_ref):
    dis = _dis_from_parts(deg_ref)
    h = jnp.dot(x_ref[...], w1_ref[...], preferred_element_type=jnp.float32)
    ms_ref[...] = h * dis[:, None]


def _tc2_body(p_ref, ms1_ref, deg_ref, w2_ref, b1_ref, ms2_ref):
    dis = _dis_from_parts(deg_ref)
    s = p_ref[0] + p_ref[1] + ms1_ref[...]
    h1 = jnp.maximum(s * dis[:, None] + b1_ref[...], 0.0)
    h2 = jnp.dot(h1, w2_ref[...], preferred_element_type=jnp.float32)
    ms2_ref[...] = h2 * dis[:, None]


def _tc3_body(p_ref, ms2_ref, deg_ref, b2_ref, wl_ref, bl_ref, out_ref,
              emb_ref):
    dis = _dis_from_parts(deg_ref)
    s = p_ref[0] + p_ref[1] + ms2_ref[...]
    emb = s * dis[:, None] + b2_ref[...]
    emb_ref[...] = emb
    logits = jnp.dot(emb, wl_ref[...], preferred_element_type=jnp.float32)
    logits = logits + bl_ref[...]
    m = jnp.max(logits, axis=1, keepdims=True)
    z = logits - m
    lse = jnp.log(jnp.sum(jnp.exp(z), axis=1, keepdims=True))
    out_ref[...] = z - lse


def kernel(x, edge_index, W1, b1, W2, b2, Wl, bl):
    npad = E_PAD - E
    pad_src = jnp.arange(npad, dtype=edge_index.dtype) % N
    srcp = jnp.concatenate([edge_index[0], pad_src])
    trash = N + jnp.arange(npad, dtype=edge_index.dtype) % N_TRASH
    dstp = jnp.concatenate([edge_index[1], trash])
    src = srcp.reshape(NW, NCHUNK, CH)
    dst = dstp.reshape(NW * NCHUNK, CH)
    dst3 = dstp.reshape(NW, NCHUNK, CH)

    deg_parts = _deg_kernel(dst3)

    grid = (N // _RB,)
    full = lambda i: (0, 0)
    rowb = lambda i: (i, 0)
    degb = lambda i: (0, i, 0)

    deg_spec = pl.BlockSpec((NC, _RB, DEG_W), degb)
    part_spec = pl.BlockSpec((NC, _RB, D), degb)
    feat_spec = pl.BlockSpec((_RB, D), rowb)

    ms1 = pl.pallas_call(
        _tc1_body,
        grid=grid,
        in_specs=[feat_spec, pl.BlockSpec((D, D), full), deg_spec],
        out_specs=feat_spec,
        out_shape=jax.ShapeDtypeStruct((N, D), jnp.float32),
    )(x, W1, deg_parts[:, :N, :])

    p1 = _spmm_kernel(ms1, src, dst)

    ms2 = pl.pallas_call(
        _tc2_body,
        grid=grid,
        in_specs=[part_spec, feat_spec, deg_spec,
                  pl.BlockSpec((D, D), full), pl.BlockSpec((1, D), full)],
        out_specs=feat_spec,
        out_shape=jax.ShapeDtypeStruct((N, D), jnp.float32),
    )(p1[:, :N, :], ms1, deg_parts[:, :N, :], W2, b1.reshape(1, D))

    p2 = _spmm_kernel(ms2, src, dst)

    out, emb = pl.pallas_call(
        _tc3_body,
        grid=grid,
        in_specs=[part_spec, feat_spec, deg_spec,
                  pl.BlockSpec((1, D), full), pl.BlockSpec((D, OUT), full),
                  pl.BlockSpec((1, OUT), full)],
        out_specs=[pl.BlockSpec((_RB, OUT), rowb), feat_spec],
        out_shape=[jax.ShapeDtypeStruct((N, OUT), jnp.float32),
                   jax.ShapeDtypeStruct((N, D), jnp.float32)],
    )(p2[:, :N, :], ms2, deg_parts[:, :N, :], b2.reshape(1, D), Wl,
      bl.reshape(1, OUT))

    return (out, emb)


# speedup vs baseline: 3.4653x; 1.2347x over previous
import functools

import jax
import jax.numpy as jnp
from jax import lax
from jax.experimental import pallas as pl
from jax.experimental.pallas import tpu as pltpu
from jax.experimental.pallas import tpu_sc as plsc

N = 10000
E = 320000
D = 128
OUT = 40

NC = 2
NS = 16
NW = NC * NS
CH = 80
NCHUNK = 128
EPW = NCHUNK * CH
E_PAD = NW * EPW
N_PAD = 10240
N_TRASH = 240
RPT = N_PAD // NS
DEG_W = 16

_mesh = plsc.VectorSubcoreMesh(core_axis_name="c", subcore_axis_name="s")


@functools.partial(
    pl.kernel,
    out_type=jax.ShapeDtypeStruct((NC, N_PAD, DEG_W), jnp.float32),
    mesh=_mesh,
    scratch_types=[
        pltpu.VMEM((NCHUNK, CH), jnp.int32),
        pltpu.VMEM((CH, DEG_W), jnp.float32),
        pltpu.VMEM_SHARED((N_PAD, DEG_W), jnp.float32),
        pltpu.SemaphoreType.DMA,
    ],
)
def _deg_kernel(dst_hbm, out_hbm, dst_v, ones_v, acc, sem):
    cid = lax.axis_index("c")
    sid = lax.axis_index("s")
    w = cid * NS + sid

    zeros16 = jnp.zeros((16,), jnp.float32)
    ones16 = jnp.ones((16,), jnp.float32)

    def zrow(i, _):
        ones_v[i, :] = zeros16
        return 0

    lax.fori_loop(0, CH, zrow, 0)
    pltpu.sync_copy(dst_hbm.at[w], dst_v)
    for k in range(RPT // CH):
        pltpu.sync_copy(ones_v, acc.at[pl.ds(sid * RPT + k * CH, CH)])

    def fill(i, _):
        ones_v[i, :] = ones16
        return 0

    lax.fori_loop(0, CH, fill, 0)
    plsc.subcore_barrier()

    def body(c, _):
        pltpu.sync_copy(ones_v, acc.at[dst_v.at[c]], add=True)
        return 0

    lax.fori_loop(0, NCHUNK, body, 0)
    plsc.subcore_barrier()
    pltpu.sync_copy(acc.at[pl.ds(sid * RPT, RPT)],
                    out_hbm.at[cid, pl.ds(sid * RPT, RPT)])


@functools.partial(
    pl.kernel,
    out_type=jax.ShapeDtypeStruct((NC, N_PAD, D), jnp.float32),
    mesh=_mesh,
    scratch_types=[
        [pltpu.VMEM((CH, D), jnp.float32) for _ in range(4)],
        [pltpu.VMEM((CH,), jnp.int32) for _ in range(8)],
        [pltpu.VMEM((CH,), jnp.int32) for _ in range(8)],
        pltpu.VMEM_SHARED((N_PAD, D), jnp.float32),
        [pltpu.SemaphoreType.DMA for _ in range(4)],
        [pltpu.SemaphoreType.DMA for _ in range(4)],
        [pltpu.SemaphoreType.DMA for _ in range(8)],
        [pltpu.SemaphoreType.DMA for _ in range(8)],
    ],
)
def _spmm_kernel(ms_hbm, src_hbm, dst_hbm, out_hbm, rbuf, sring, dring, acc,
                 semg, sems, semsr, semid):
    cid = lax.axis_index("c")
    sid = lax.axis_index("s")
    w = cid * NS + sid
    wbase = w * NCHUNK

    zeros16 = jnp.zeros((16,), jnp.float32)

    def zrow(i, _):
        for j in range(D // 16):
            rbuf[0][i, pl.ds(j * 16, 16)] = zeros16
        return 0

    lax.fori_loop(0, CH, zrow, 0)
    for k in range(RPT // CH):
        pltpu.sync_copy(rbuf[0], acc.at[pl.ds(sid * RPT + k * CH, CH)])
    plsc.subcore_barrier()

    def fetch_idx(c, k):
        pltpu.async_copy(src_hbm.at[wbase + c], sring[k], semsr[k])
        pltpu.async_copy(dst_hbm.at[wbase + c], dring[k], semid[k])

    def wait_src_idx(c, k):
        pltpu.make_async_copy(src_hbm.at[wbase + c], sring[k],
                              semsr[k]).wait()

    def wait_dst_idx(c, k):
        pltpu.make_async_copy(dst_hbm.at[wbase + c], dring[k],
                              semid[k]).wait()

    def start_gather(k, b):
        pltpu.async_copy(ms_hbm.at[sring[k]], rbuf[b], semg[b])

    def wait_gather(k, b):
        pltpu.make_async_copy(ms_hbm.at[sring[k]], rbuf[b], semg[b]).wait()

    def start_scatter(k, b):
        pltpu.async_copy(rbuf[b], acc.at[dring[k]], sems[b], add=True)

    def wait_scatter(k, b):
        pltpu.make_async_copy(rbuf[b], acc.at[dring[k]], sems[b]).wait()

    def step(c, j, do_wait_scatter=True, do_fetch=True, do_gather=True):
        if do_wait_scatter:
            wait_scatter((j - 2) % 8, (j - 2) % 4)
        if do_fetch:
            fetch_idx(c + 6, (j + 6) % 8)
        wait_gather(j % 8, j % 4)
        if do_gather:
            wait_src_idx(c + 2, (j + 2) % 8)
            start_gather((j + 2) % 8, (j + 2) % 4)
        wait_dst_idx(c, j % 8)
        start_scatter(j % 8, j % 4)

    for c in range(6):
        fetch_idx(c, c)
    for c in range(2):
        wait_src_idx(c, c)
        start_gather(c, c)
    step(0, 0, do_wait_scatter=False)
    step(1, 1, do_wait_scatter=False)
    for c in range(2, 8):
        step(c, c)

    def q_body(q, _):
        c = 8 * q
        for j in range(8):
            step(c + j, j)
        return 0

    lax.fori_loop(1, (NCHUNK - 8) // 8, q_body, 0)
    for c in range(NCHUNK - 8, NCHUNK):
        step(c, c % 8, do_fetch=(c + 6 < NCHUNK),
             do_gather=(c + 2 < NCHUNK))
    wait_scatter((NCHUNK - 2) % 8, (NCHUNK - 2) % 4)
    wait_scatter((NCHUNK - 1) % 8, (NCHUNK - 1) % 4)
    plsc.subcore_barrier()
    pltpu.sync_copy(acc.at[pl.ds(sid * RPT, RPT)],
                    out_hbm.at[cid, pl.ds(sid * RPT, RPT)])


_RB = 1000


def _dis_from_parts(deg_ref):
    deg = deg_ref[0, :, 0] + deg_ref[1, :, 0] + 1.0
    return lax.rsqrt(deg)


def _tc1_body(x_ref, w1_ref, deg_ref, ms_ref):
    dis = _dis_from_parts(deg_ref)
    h = jnp.dot(x_ref[...], w1_ref[...], preferred_element_type=jnp.float32)
    ms_ref[...] = h * dis[:, None]


def _tc2_body(p_ref, ms1_ref, deg_ref, w2_ref, b1_ref, ms2_ref):
    dis = _dis_from_parts(deg_ref)
    s = p_ref[0] + p_ref[1] + ms1_ref[...]
    h1 = jnp.maximum(s * dis[:, None] + b1_ref[...], 0.0)
    h2 = jnp.dot(h1, w2_ref[...], preferred_element_type=jnp.float32)
    ms2_ref[...] = h2 * dis[:, None]


def _tc3_body(p_ref, ms2_ref, deg_ref, b2_ref, wl_ref, bl_ref, out_ref,
              emb_ref):
    dis = _dis_from_parts(deg_ref)
    s = p_ref[0] + p_ref[1] + ms2_ref[...]
    emb = s * dis[:, None] + b2_ref[...]
    emb_ref[...] = emb
    logits = jnp.dot(emb, wl_ref[...], preferred_element_type=jnp.float32)
    logits = logits + bl_ref[...]
    m = jnp.max(logits, axis=1, keepdims=True)
    z = logits - m
    lse = jnp.log(jnp.sum(jnp.exp(z), axis=1, keepdims=True))
    out_ref[...] = z - lse


def kernel(x, edge_index, W1, b1, W2, b2, Wl, bl):
    npad = E_PAD - E
    pad_src = jnp.arange(npad, dtype=edge_index.dtype) % N
    srcp = jnp.concatenate([edge_index[0], pad_src])
    trash = N + jnp.arange(npad, dtype=edge_index.dtype) % N_TRASH
    dstp = jnp.concatenate([edge_index[1], trash])
    src = srcp.reshape(NW * NCHUNK, CH)
    dst = dstp.reshape(NW * NCHUNK, CH)
    dst3 = dstp.reshape(NW, NCHUNK, CH)

    deg_parts = _deg_kernel(dst3)

    grid = (N // _RB,)
    full = lambda i: (0, 0)
    rowb = lambda i: (i, 0)
    degb = lambda i: (0, i, 0)

    deg_spec = pl.BlockSpec((NC, _RB, DEG_W), degb)
    part_spec = pl.BlockSpec((NC, _RB, D), degb)
    feat_spec = pl.BlockSpec((_RB, D), rowb)

    ms1 = pl.pallas_call(
        _tc1_body,
        grid=grid,
        in_specs=[feat_spec, pl.BlockSpec((D, D), full), deg_spec],
        out_specs=feat_spec,
        out_shape=jax.ShapeDtypeStruct((N, D), jnp.float32),
    )(x, W1, deg_parts[:, :N, :])

    p1 = _spmm_kernel(ms1, src, dst)

    ms2 = pl.pallas_call(
        _tc2_body,
        grid=grid,
        in_specs=[part_spec, feat_spec, deg_spec,
                  pl.BlockSpec((D, D), full), pl.BlockSpec((1, D), full)],
        out_specs=feat_spec,
        out_shape=jax.ShapeDtypeStruct((N, D), jnp.float32),
    )(p1[:, :N, :], ms1, deg_parts[:, :N, :], W2, b1.reshape(1, D))

    p2 = _spmm_kernel(ms2, src, dst)

    out, emb = pl.pallas_call(
        _tc3_body,
        grid=grid,
        in_specs=[part_spec, feat_spec, deg_spec,
                  pl.BlockSpec((1, D), full), pl.BlockSpec((D, OUT), full),
                  pl.BlockSpec((1, OUT), full)],
        out_specs=[pl.BlockSpec((_RB, OUT), rowb), feat_spec],
        out_shape=[jax.ShapeDtypeStruct((N, OUT), jnp.float32),
                   jax.ShapeDtypeStruct((N, D), jnp.float32)],
    )(p2[:, :N, :], ms2, deg_parts[:, :N, :], b2.reshape(1, D), Wl,
      bl.reshape(1, OUT))

    return (out, emb)


# speedup vs baseline: 3.5327x; 1.0194x over previous
import functools

import jax
import jax.numpy as jnp
from jax import lax
from jax.experimental import pallas as pl
from jax.experimental.pallas import tpu as pltpu
from jax.experimental.pallas import tpu_sc as plsc

N = 10000
E = 320000
D = 128
OUT = 40

NC = 2
NS = 16
NW = NC * NS
CH = 80
NCHUNK = 128
EPW = NCHUNK * CH
E_PAD = NW * EPW
N_PAD = 10240
N_TRASH = 240
RPT = N_PAD // NS
DEG_W = 16

_mesh = plsc.VectorSubcoreMesh(core_axis_name="c", subcore_axis_name="s")


@functools.partial(
    pl.kernel,
    out_type=jax.ShapeDtypeStruct((NC, N_PAD, DEG_W), jnp.float32),
    mesh=_mesh,
    scratch_types=[
        pltpu.VMEM((NCHUNK, CH), jnp.int32),
        pltpu.VMEM((CH, DEG_W), jnp.float32),
        pltpu.VMEM_SHARED((N_PAD, DEG_W), jnp.float32),
        [pltpu.SemaphoreType.DMA for _ in range(4)],
    ],
)
def _deg_kernel(dst_hbm, out_hbm, dst_v, ones_v, acc, sem):
    cid = lax.axis_index("c")
    sid = lax.axis_index("s")
    w = cid * NS + sid

    zeros16 = jnp.zeros((16,), jnp.float32)
    ones16 = jnp.ones((16,), jnp.float32)

    def zrow(i, _):
        ones_v[i, :] = zeros16
        return 0

    lax.fori_loop(0, CH, zrow, 0)
    pltpu.sync_copy(dst_hbm.at[w], dst_v)
    for k in range(RPT // CH):
        pltpu.sync_copy(ones_v, acc.at[pl.ds(sid * RPT + k * CH, CH)])

    def fill(i, _):
        ones_v[i, :] = ones16
        return 0

    lax.fori_loop(0, CH, fill, 0)
    plsc.subcore_barrier()

    def dscat(c, j):
        pltpu.async_copy(ones_v, acc.at[dst_v.at[c]], sem[j], add=True)

    def dwait(c, j):
        pltpu.make_async_copy(ones_v, acc.at[dst_v.at[c]], sem[j]).wait()

    for j in range(4):
        dscat(j, j)

    def body(q, _):
        c = 4 * q
        for j in range(4):
            dwait(c + j - 4, j)
            dscat(c + j, j)
        return 0

    lax.fori_loop(1, NCHUNK // 4, body, 0)
    for j in range(4):
        dwait(NCHUNK - 4 + j, j)
    plsc.subcore_barrier()
    pltpu.sync_copy(acc.at[pl.ds(sid * RPT, RPT)],
                    out_hbm.at[cid, pl.ds(sid * RPT, RPT)])


@functools.partial(
    pl.kernel,
    out_type=jax.ShapeDtypeStruct((NC, N_PAD, D), jnp.float32),
    mesh=_mesh,
    scratch_types=[
        [pltpu.VMEM((CH, D), jnp.float32) for _ in range(4)],
        [pltpu.VMEM((CH,), jnp.int32) for _ in range(8)],
        [pltpu.VMEM((CH,), jnp.int32) for _ in range(8)],
        pltpu.VMEM_SHARED((N_PAD, D), jnp.float32),
        [pltpu.SemaphoreType.DMA for _ in range(4)],
        [pltpu.SemaphoreType.DMA for _ in range(4)],
        [pltpu.SemaphoreType.DMA for _ in range(8)],
        [pltpu.SemaphoreType.DMA for _ in range(8)],
    ],
)
def _spmm_kernel(ms_hbm, src_hbm, dst_hbm, out_hbm, rbuf, sring, dring, acc,
                 semg, sems, semsr, semid):
    cid = lax.axis_index("c")
    sid = lax.axis_index("s")
    w = cid * NS + sid
    wbase = w * NCHUNK

    zeros16 = jnp.zeros((16,), jnp.float32)

    def zrow(i, _):
        for j in range(D // 16):
            rbuf[0][i, pl.ds(j * 16, 16)] = zeros16
        return 0

    lax.fori_loop(0, CH, zrow, 0)
    for k in range(RPT // CH):
        pltpu.sync_copy(rbuf[0], acc.at[pl.ds(sid * RPT + k * CH, CH)])
    plsc.subcore_barrier()

    def fetch_idx(c, k):
        pltpu.async_copy(src_hbm.at[wbase + c], sring[k], semsr[k])
        pltpu.async_copy(dst_hbm.at[wbase + c], dring[k], semid[k])

    def wait_src_idx(c, k):
        pltpu.make_async_copy(src_hbm.at[wbase + c], sring[k],
                              semsr[k]).wait()

    def wait_dst_idx(c, k):
        pltpu.make_async_copy(dst_hbm.at[wbase + c], dring[k],
                              semid[k]).wait()

    def start_gather(k, b):
        pltpu.async_copy(ms_hbm.at[sring[k]], rbuf[b], semg[b])

    def wait_gather(k, b):
        pltpu.make_async_copy(ms_hbm.at[sring[k]], rbuf[b], semg[b]).wait()

    def start_scatter(k, b):
        pltpu.async_copy(rbuf[b], acc.at[dring[k]], sems[b], add=True)

    def wait_scatter(k, b):
        pltpu.make_async_copy(rbuf[b], acc.at[dring[k]], sems[b]).wait()

    def step(c, j, do_wait_scatter=True, do_fetch=True, do_gather=True):
        if do_wait_scatter:
            wait_scatter((j - 2) % 8, (j - 2) % 4)
        if do_fetch:
            fetch_idx(c + 6, (j + 6) % 8)
        wait_gather(j % 8, j % 4)
        if do_gather:
            wait_src_idx(c + 2, (j + 2) % 8)
            start_gather((j + 2) % 8, (j + 2) % 4)
        wait_dst_idx(c, j % 8)
        start_scatter(j % 8, j % 4)

    for c in range(6):
        fetch_idx(c, c)
    for c in range(2):
        wait_src_idx(c, c)
        start_gather(c, c)
    step(0, 0, do_wait_scatter=False)
    step(1, 1, do_wait_scatter=False)
    for c in range(2, 8):
        step(c, c)

    def q_body(q, _):
        c = 8 * q
        for j in range(8):
            step(c + j, j)
        return 0

    lax.fori_loop(1, (NCHUNK - 8) // 8, q_body, 0)
    for c in range(NCHUNK - 8, NCHUNK):
        step(c, c % 8, do_fetch=(c + 6 < NCHUNK),
             do_gather=(c + 2 < NCHUNK))
    wait_scatter((NCHUNK - 2) % 8, (NCHUNK - 2) % 4)
    wait_scatter((NCHUNK - 1) % 8, (NCHUNK - 1) % 4)
    plsc.subcore_barrier()
    pltpu.sync_copy(acc.at[pl.ds(sid * RPT, RPT)],
                    out_hbm.at[cid, pl.ds(sid * RPT, RPT)])


_RB = 1000


def _dis_from_parts(deg_ref):
    deg = deg_ref[0, :, 0] + deg_ref[1, :, 0] + 1.0
    return lax.rsqrt(deg)


def _tc1_body(x_ref, w1_ref, deg_ref, ms_ref):
    dis = _dis_from_parts(deg_ref)
    h = jnp.dot(x_ref[...], w1_ref[...], preferred_element_type=jnp.float32)
    ms_ref[...] = h * dis[:, None]


def _tc2_body(p_ref, ms1_ref, deg_ref, w2_ref, b1_ref, ms2_ref):
    dis = _dis_from_parts(deg_ref)
    s = p_ref[0] + p_ref[1] + ms1_ref[...]
    h1 = jnp.maximum(s * dis[:, None] + b1_ref[...], 0.0)
    h2 = jnp.dot(h1, w2_ref[...], preferred_element_type=jnp.float32)
    ms2_ref[...] = h2 * dis[:, None]


def _tc3_body(p_ref, ms2_ref, deg_ref, b2_ref, wl_ref, bl_ref, out_ref,
              emb_ref):
    dis = _dis_from_parts(deg_ref)
    s = p_ref[0] + p_ref[1] + ms2_ref[...]
    emb = s * dis[:, None] + b2_ref[...]
    emb_ref[...] = emb
    logits = jnp.dot(emb, wl_ref[...], preferred_element_type=jnp.float32)
    logits = logits + bl_ref[...]
    m = jnp.max(logits, axis=1, keepdims=True)
    z = logits - m
    lse = jnp.log(jnp.sum(jnp.exp(z), axis=1, keepdims=True))
    out_ref[...] = z - lse


def kernel(x, edge_index, W1, b1, W2, b2, Wl, bl):
    npad = E_PAD - E
    pad_src = jnp.arange(npad, dtype=edge_index.dtype) % N
    srcp = jnp.concatenate([edge_index[0], pad_src])
    trash = N + jnp.arange(npad, dtype=edge_index.dtype) % N_TRASH
    dstp = jnp.concatenate([edge_index[1], trash])
    src = srcp.reshape(NW * NCHUNK, CH)
    dst = dstp.reshape(NW * NCHUNK, CH)
    dst3 = dstp.reshape(NW, NCHUNK, CH)

    deg_parts = _deg_kernel(dst3)

    grid = (N // _RB,)
    full = lambda i: (0, 0)
    rowb = lambda i: (i, 0)
    degb = lambda i: (0, i, 0)

    deg_spec = pl.BlockSpec((NC, _RB, DEG_W), degb)
    part_spec = pl.BlockSpec((NC, _RB, D), degb)
    feat_spec = pl.BlockSpec((_RB, D), rowb)

    ms1 = pl.pallas_call(
        _tc1_body,
        grid=grid,
        in_specs=[feat_spec, pl.BlockSpec((D, D), full), deg_spec],
        out_specs=feat_spec,
        out_shape=jax.ShapeDtypeStruct((N, D), jnp.float32),
    )(x, W1, deg_parts[:, :N, :])

    p1 = _spmm_kernel(ms1, src, dst)

    ms2 = pl.pallas_call(
        _tc2_body,
        grid=grid,
        in_specs=[part_spec, feat_spec, deg_spec,
                  pl.BlockSpec((D, D), full), pl.BlockSpec((1, D), full)],
        out_specs=feat_spec,
        out_shape=jax.ShapeDtypeStruct((N, D), jnp.float32),
    )(p1[:, :N, :], ms1, deg_parts[:, :N, :], W2, b1.reshape(1, D))

    p2 = _spmm_kernel(ms2, src, dst)

    out, emb = pl.pallas_call(
        _tc3_body,
        grid=grid,
        in_specs=[part_spec, feat_spec, deg_spec,
                  pl.BlockSpec((1, D), full), pl.BlockSpec((D, OUT), full),
                  pl.BlockSpec((1, OUT), full)],
        out_specs=[pl.BlockSpec((_RB, OUT), rowb), feat_spec],
        out_shape=[jax.ShapeDtypeStruct((N, OUT), jnp.float32),
                   jax.ShapeDtypeStruct((N, D), jnp.float32)],
    )(p2[:, :N, :], ms2, deg_parts[:, :N, :], b2.reshape(1, D), Wl,
      bl.reshape(1, OUT))

    return (out, emb)
